# baseline (speedup 1.0000x reference)
import jax
import jax.numpy as jnp
from jax import lax
from jax.experimental import pallas as pl
from jax.experimental.pallas import tpu as pltpu

_B, _D, _H, _W = 8, 256, 32, 32
_M = 1024
_HW = _H * _W
_N = _B * _HW
_T = 256
_NT = _N // _T
_TPB = _HW // _T
_NEG = -1e30


def _qn_t(qt):
    n2 = jnp.sum(qt * qt, axis=0, keepdims=True)
    return qt * lax.rsqrt(jnp.maximum(n2, 1e-24))


def _score_t(qnt, keys):
    return lax.dot_general(qnt, keys, (((0,), (1,)), ((), ())),
                           preferred_element_type=jnp.float32)


def _split_hi_lo(x):
    hi = x.astype(jnp.bfloat16).astype(jnp.float32)
    return hi, x - hi


def _stats_kernel(qt_ref, keys_ref, cs_ref, cme_ref, kn2_ref):
    i = pl.program_id(0)
    keys = keys_ref[...]
    escore = jnp.exp(_score_t(_qn_t(qt_ref[0]), keys))

    @pl.when(i == 0)
    def _():
        cs_ref[...] = jnp.zeros((1, _M), jnp.float32)
        cme_ref[...] = jnp.zeros((1, _M), jnp.float32)
        ones_row = jnp.ones((1, _D), jnp.float32)
        kn2_ref[...] = lax.dot_general(ones_row, keys * keys,
                                       (((1,), (1,)), ((), ())),
                                       preferred_element_type=jnp.float32)

    cs_ref[...] += jnp.sum(escore, axis=0, keepdims=True)
    cme_ref[...] = jnp.maximum(cme_ref[...], jnp.max(escore, axis=0, keepdims=True))


def _main_kernel(qt_ref, keys_ref, cs_ref, cme_ref, kn2_ref,
                 sm_ref, sq_ref, uq_ref, g_ref, s_ref, upd_ref,
                 gscr, sscr, uscr):
    i = pl.program_id(0)
    keys = keys_ref[...]
    qnt = _qn_t(qt_ref[0])
    uq_ref[0, :_D, :] = qnt
    score = _score_t(qnt, keys)

    rmax = jnp.max(score, axis=1, keepdims=True)
    e = jnp.exp(score - rmax)
    rsum = jnp.sum(e, axis=1, keepdims=True)
    sm = e * (1.0 / rsum)
    sm_ref[...] = sm

    u = jnp.exp(rmax)
    v = 1.0 / cs_ref[...]
    sq_ref[...] = e * u * v

    uq_ref[0, _D:, :] = lax.dot_general(keys, sm, (((0,), (1,)), ((), ())),
                                        preferred_element_type=jnp.float32)

    b1 = score >= rmax
    oh1 = b1.astype(jnp.float32)
    masked = jnp.where(b1, _NEG, score)
    m2 = jnp.max(masked, axis=1, keepdims=True)
    oh2 = (masked >= m2).astype(jnp.float32)

    cme_hi, cme_lo = _split_hi_lo(cme_ref[...])
    vrows = jnp.concatenate([cme_hi, cme_lo, kn2_ref[...]], axis=0)
    g1 = lax.dot_general(oh1, vrows, (((1,), (1,)), ((), ())),
                         preferred_element_type=jnp.float32)
    cme_g = g1[:, 0:1] + g1[:, 1:2]
    kn2_g = g1[:, 2:3]
    kn2_g2 = lax.dot_general(oh2, kn2_ref[...], (((1,), (1,)), ((), ())),
                             preferred_element_type=jnp.float32)

    @pl.when(i == 0)
    def _():
        gscr[...] = jnp.zeros((_B, 1), jnp.float32)
        sscr[...] = jnp.zeros((_B, 1), jnp.float32)
        uscr[...] = jnp.zeros((_M, _D), jnp.float32)

    boh = lax.broadcasted_iota(jnp.int32, (_B, 1), 0) == (i // _TPB)

    d1sq = 1.0 - 2.0 * rmax + kn2_g
    gscr[...] += jnp.where(boh, jnp.sum(d1sq) / (_HW * _D * 1.0), 0.0)

    d2sq = 1.0 - 2.0 * m2 + kn2_g2
    dp = jnp.sqrt(jnp.maximum(d1sq, 0.0))
    dn = jnp.sqrt(jnp.maximum(d2sq, 0.0))
    s_row = jnp.maximum(dp - dn + 1.0, 0.0)
    sscr[...] += jnp.where(boh, jnp.sum(s_row) / (_HW * 1.0), 0.0)

    wgt = u * (1.0 / cme_g)
    wm = oh1 * wgt
    uscr[...] += lax.dot_general(wm, qnt, (((0,), (1,)), ((), ())),
                                 preferred_element_type=jnp.float32)

    @pl.when(i == _NT - 1)
    def _():
        g_ref[...] = gscr[...]
        s_ref[...] = sscr[...]
        upd = uscr[...] + keys
        n2 = jnp.sum(upd * upd, axis=1, keepdims=True)
        upd_ref[...] = upd * lax.rsqrt(jnp.maximum(n2, 1e-24))


def kernel(query, keys):
    qt = query.reshape(_B, _D, _HW)
    f32 = jnp.float32

    cs, cme, kn2 = pl.pallas_call(
        _stats_kernel,
        grid=(_NT,),
        in_specs=[
            pl.BlockSpec((1, _D, _T), lambda i: (i // _TPB, 0, i % _TPB)),
            pl.BlockSpec((_M, _D), lambda i: (0, 0)),
        ],
        out_specs=[
            pl.BlockSpec((1, _M), lambda i: (0, 0)),
            pl.BlockSpec((1, _M), lambda i: (0, 0)),
            pl.BlockSpec((1, _M), lambda i: (0, 0)),
        ],
        out_shape=[
            jax.ShapeDtypeStruct((1, _M), f32),
            jax.ShapeDtypeStruct((1, _M), f32),
            jax.ShapeDtypeStruct((1, _M), f32),
        ],
    )(qt, keys)

    sm, sq, uq, g_loss, s_loss, upd = pl.pallas_call(
        _main_kernel,
        grid=(_NT,),
        in_specs=[
            pl.BlockSpec((1, _D, _T), lambda i: (i // _TPB, 0, i % _TPB)),
            pl.BlockSpec((_M, _D), lambda i: (0, 0)),
            pl.BlockSpec((1, _M), lambda i: (0, 0)),
            pl.BlockSpec((1, _M), lambda i: (0, 0)),
            pl.BlockSpec((1, _M), lambda i: (0, 0)),
        ],
        out_specs=[
            pl.BlockSpec((_T, _M), lambda i: (i, 0)),
            pl.BlockSpec((_T, _M), lambda i: (i, 0)),
            pl.BlockSpec((1, 2 * _D, _T), lambda i: (i // _TPB, 0, i % _TPB)),
            pl.BlockSpec((_B, 1), lambda i: (0, 0)),
            pl.BlockSpec((_B, 1), lambda i: (0, 0)),
            pl.BlockSpec((_M, _D), lambda i: (0, 0)),
        ],
        out_shape=[
            jax.ShapeDtypeStruct((_N, _M), f32),
            jax.ShapeDtypeStruct((_N, _M), f32),
            jax.ShapeDtypeStruct((_B, 2 * _D, _HW), f32),
            jax.ShapeDtypeStruct((_B, 1), f32),
            jax.ShapeDtypeStruct((_B, 1), f32),
            jax.ShapeDtypeStruct((_M, _D), f32),
        ],
        scratch_shapes=[
            pltpu.VMEM((_B, 1), f32),
            pltpu.VMEM((_B, 1), f32),
            pltpu.VMEM((_M, _D), f32),
        ],
    )(qt, keys, cs, cme, kn2)

    return (uq.reshape(_B, 2 * _D, _H, _W), upd, sq, sm, g_loss, s_loss)


# speedup vs baseline: 1.2909x; 1.2909x over previous
import jax
import jax.numpy as jnp
from jax import lax
from jax.experimental import pallas as pl
from jax.experimental.pallas import tpu as pltpu

_B, _D, _H, _W = 8, 256, 32, 32
_M = 1024
_HW = _H * _W
_N = _B * _HW
_T = 256
_NT = _N // _T
_TPB = _HW // _T
_NEG = -1e30


def _norm_rows(q):
    n2 = jnp.sum(q * q, axis=1, keepdims=True)
    return q * lax.rsqrt(jnp.maximum(n2, 1e-24))


def _score_of(qn, keys):
    return lax.dot_general(qn, keys, (((1,), (1,)), ((), ())),
                           preferred_element_type=jnp.float32)


def _split_hi_lo(x):
    hi = x.astype(jnp.bfloat16).astype(jnp.float32)
    return hi, x - hi


def _stats_kernel(qf_ref, keys_ref, cs_ref, cme_ref, kn2_ref):
    i = pl.program_id(0)
    keys = keys_ref[...]
    escore = jnp.exp(_score_of(_norm_rows(qf_ref[...]), keys))

    @pl.when(i == 0)
    def _():
        cs_ref[...] = jnp.zeros((1, _M), jnp.float32)
        cme_ref[...] = jnp.zeros((1, _M), jnp.float32)
        ones_row = jnp.ones((1, _D), jnp.float32)
        kn2_ref[...] = lax.dot_general(ones_row, keys * keys,
                                       (((1,), (1,)), ((), ())),
                                       preferred_element_type=jnp.float32)

    cs_ref[...] += jnp.sum(escore, axis=0, keepdims=True)
    cme_ref[...] = jnp.maximum(cme_ref[...], jnp.max(escore, axis=0, keepdims=True))


def _main_kernel(qf_ref, keys_ref, cs_ref, cme_ref, kn2_ref,
                 sm_ref, sq_ref, qcat_ref, g_ref, s_ref, upd_ref,
                 gscr, sscr, uscr):
    i = pl.program_id(0)
    keys = keys_ref[...]
    qn = _norm_rows(qf_ref[...])
    qcat_ref[:, :_D] = qn
    score = _score_of(qn, keys)

    rmax = jnp.max(score, axis=1, keepdims=True)
    e = jnp.exp(score - rmax)
    rsum = jnp.sum(e, axis=1, keepdims=True)
    sm = e * (1.0 / rsum)
    sm_ref[...] = sm

    u = jnp.exp(rmax)
    v = 1.0 / cs_ref[...]
    sq_ref[...] = e * u * v

    qcat_ref[:, _D:] = lax.dot_general(sm, keys, (((1,), (0,)), ((), ())),
                                       preferred_element_type=jnp.float32)

    b1 = score >= rmax
    oh1 = b1.astype(jnp.float32)
    masked = jnp.where(b1, _NEG, score)
    m2 = jnp.max(masked, axis=1, keepdims=True)
    oh2 = (masked >= m2).astype(jnp.float32)

    cme_hi, cme_lo = _split_hi_lo(cme_ref[...])
    vrows = jnp.concatenate([cme_hi, cme_lo, kn2_ref[...]], axis=0)
    g1 = lax.dot_general(oh1, vrows, (((1,), (1,)), ((), ())),
                         preferred_element_type=jnp.float32)
    cme_g = g1[:, 0:1] + g1[:, 1:2]
    kn2_g = g1[:, 2:3]
    kn2_g2 = lax.dot_general(oh2, kn2_ref[...], (((1,), (1,)), ((), ())),
                             preferred_element_type=jnp.float32)

    @pl.when(i == 0)
    def _():
        gscr[...] = jnp.zeros((_B, 1), jnp.float32)
        sscr[...] = jnp.zeros((_B, 1), jnp.float32)
        uscr[...] = jnp.zeros((_M, _D), jnp.float32)

    boh = lax.broadcasted_iota(jnp.int32, (_B, 1), 0) == (i // _TPB)

    d1sq = 1.0 - 2.0 * rmax + kn2_g
    gscr[...] += jnp.where(boh, jnp.sum(d1sq) / (_HW * _D * 1.0), 0.0)

    d2sq = 1.0 - 2.0 * m2 + kn2_g2
    dp = jnp.sqrt(jnp.maximum(d1sq, 0.0))
    dn = jnp.sqrt(jnp.maximum(d2sq, 0.0))
    s_row = jnp.maximum(dp - dn + 1.0, 0.0)
    sscr[...] += jnp.where(boh, jnp.sum(s_row) / (_HW * 1.0), 0.0)

    wgt = u * (1.0 / cme_g)
    wm = oh1 * wgt
    uscr[...] += lax.dot_general(wm, qn, (((0,), (0,)), ((), ())),
                                 preferred_element_type=jnp.float32)

    @pl.when(i == _NT - 1)
    def _():
        g_ref[...] = gscr[...]
        s_ref[...] = sscr[...]
        upd = uscr[...] + keys
        n2 = jnp.sum(upd * upd, axis=1, keepdims=True)
        upd_ref[...] = upd * lax.rsqrt(jnp.maximum(n2, 1e-24))


def kernel(query, keys):
    qf = jnp.transpose(query, (0, 2, 3, 1)).reshape(_N, _D)
    f32 = jnp.float32

    cs, cme, kn2 = pl.pallas_call(
        _stats_kernel,
        grid=(_NT,),
        in_specs=[
            pl.BlockSpec((_T, _D), lambda i: (i, 0)),
            pl.BlockSpec((_M, _D), lambda i: (0, 0)),
        ],
        out_specs=[
            pl.BlockSpec((1, _M), lambda i: (0, 0)),
            pl.BlockSpec((1, _M), lambda i: (0, 0)),
            pl.BlockSpec((1, _M), lambda i: (0, 0)),
        ],
        out_shape=[
            jax.ShapeDtypeStruct((1, _M), f32),
            jax.ShapeDtypeStruct((1, _M), f32),
            jax.ShapeDtypeStruct((1, _M), f32),
        ],
    )(qf, keys)

    sm, sq, qcat, g_loss, s_loss, upd = pl.pallas_call(
        _main_kernel,
        grid=(_NT,),
        in_specs=[
            pl.BlockSpec((_T, _D), lambda i: (i, 0)),
            pl.BlockSpec((_M, _D), lambda i: (0, 0)),
            pl.BlockSpec((1, _M), lambda i: (0, 0)),
            pl.BlockSpec((1, _M), lambda i: (0, 0)),
            pl.BlockSpec((1, _M), lambda i: (0, 0)),
        ],
        out_specs=[
            pl.BlockSpec((_T, _M), lambda i: (i, 0)),
            pl.BlockSpec((_T, _M), lambda i: (i, 0)),
            pl.BlockSpec((_T, 2 * _D), lambda i: (i, 0)),
            pl.BlockSpec((_B, 1), lambda i: (0, 0)),
            pl.BlockSpec((_B, 1), lambda i: (0, 0)),
            pl.BlockSpec((_M, _D), lambda i: (0, 0)),
        ],
        out_shape=[
            jax.ShapeDtypeStruct((_N, _M), f32),
            jax.ShapeDtypeStruct((_N, _M), f32),
            jax.ShapeDtypeStruct((_N, 2 * _D), f32),
            jax.ShapeDtypeStruct((_B, 1), f32),
            jax.ShapeDtypeStruct((_B, 1), f32),
            jax.ShapeDtypeStruct((_M, _D), f32),
        ],
        scratch_shapes=[
            pltpu.VMEM((_B, 1), f32),
            pltpu.VMEM((_B, 1), f32),
            pltpu.VMEM((_M, _D), f32),
        ],
    )(qf, keys, cs, cme, kn2)

    uq = qcat.reshape(_B, _H, _W, 2 * _D).transpose(0, 3, 1, 2)
    return (uq, upd, sq, sm, g_loss, s_loss)
